# baseline (speedup 1.0000x reference)
import functools

import jax
import jax.numpy as jnp
from jax import lax
from jax.experimental import pallas as pl
from jax.experimental.pallas import tpu as pltpu
from jax.experimental.pallas import tpu_sc as plsc

_NC = 2
_NS = 16
_NW = _NC * _NS
_CH = 128
_ROWB = 1024


def _rup(x, m):
    return (x + m - 1) // m * m


_NBUF = 4


def _sc_scatter(src_i, dst_i, table, ar):
    et = src_i.shape[0]
    d = table.shape[1]
    epw = et // _NW
    steps = epw // _CH
    ngroups = steps // _NBUF
    zr = ar // _NS
    zfull, ztail = zr // _CH, zr % _CH
    mesh = plsc.VectorSubcoreMesh(core_axis_name="c", subcore_axis_name="s")

    @functools.partial(
        pl.kernel,
        out_type=jax.ShapeDtypeStruct((_NC, ar, d), jnp.float32),
        mesh=mesh,
        scratch_types=[
            [pltpu.VMEM((_CH,), jnp.int32)] * _NBUF,
            [pltpu.VMEM((_CH,), jnp.int32)] * _NBUF,
            [pltpu.VMEM((_CH, d), jnp.float32)] * 2,
            pltpu.VMEM_SHARED((ar, d), jnp.float32),
            [pltpu.SemaphoreType.DMA] * _NBUF,
            [pltpu.SemaphoreType.DMA] * _NBUF,
            [pltpu.SemaphoreType.DMA] * 2,
        ],
    )
    def k(zeros_hbm, src_hbm, dst_hbm, table_hbm, out_hbm,
          idx_s, idx_d, rows, acc, sem_is, sem_id, sem_g):
        c = lax.axis_index("c")
        s = lax.axis_index("s")
        wid = s * _NC + c
        base = pl.multiple_of(wid * epw, _CH)
        pltpu.sync_copy(zeros_hbm, rows[0])
        zb = s * zr
        for j in range(zfull):
            pltpu.sync_copy(rows[0], acc.at[pl.ds(zb + j * _CH, _CH)])
        if ztail:
            pltpu.sync_copy(rows[0].at[pl.ds(0, ztail)],
                            acc.at[pl.ds(zb + zfull * _CH, ztail)])
        plsc.subcore_barrier()

        def start_idx(dstbuf, hbm, ch, sem):
            off = pl.multiple_of(base + ch * _CH, _CH)
            pltpu.async_copy(hbm.at[pl.ds(off, _CH)], dstbuf, sem)

        def wait_idx(dstbuf, hbm, sem):
            pltpu.make_async_copy(hbm.at[pl.ds(base, _CH)], dstbuf, sem).wait()

        for q in range(_NBUF):
            start_idx(idx_s[q], src_hbm, q, sem_is[q])
            start_idx(idx_d[q], dst_hbm, q, sem_id[q])
        for r in range(2):
            wait_idx(idx_s[r], src_hbm, sem_is[r])
            pltpu.async_copy(table_hbm.at[idx_s[r]], rows[r], sem_g[r])

        def group(g, carry):
            ch0 = g * _NBUF
            for bb in range(_NBUF):
                ch = ch0 + bb
                r = bb % 2
                q = bb
                pltpu.make_async_copy(table_hbm.at[idx_s[q]],
                                      rows[r], sem_g[r]).wait()

                @pl.when(ch + _NBUF < steps)
                def _():
                    start_idx(idx_s[q], src_hbm, ch + _NBUF, sem_is[q])

                wait_idx(idx_d[q], dst_hbm, sem_id[q])
                pltpu.sync_copy(rows[r], acc.at[idx_d[q]], add=True)

                @pl.when(ch + _NBUF < steps)
                def _():
                    start_idx(idx_d[q], dst_hbm, ch + _NBUF, sem_id[q])

                @pl.when(ch + 2 < steps)
                def _():
                    q2 = (bb + 2) % _NBUF
                    wait_idx(idx_s[q2], src_hbm, sem_is[q2])
                    pltpu.async_copy(table_hbm.at[idx_s[q2]], rows[r],
                                     sem_g[r])
            return carry

        lax.fori_loop(0, ngroups, group, 0)
        plsc.subcore_barrier()

        for cc in range(_NC):
            @pl.when(c == cc)
            def _():
                for j in range(zfull):
                    pltpu.sync_copy(acc.at[pl.ds(zb + j * _CH, _CH)],
                                    out_hbm.at[cc, pl.ds(zb + j * _CH, _CH)])
                if ztail:
                    pltpu.sync_copy(acc.at[pl.ds(zb + zfull * _CH, ztail)],
                                    out_hbm.at[cc, pl.ds(zb + zfull * _CH, ztail)])

    zeros = jnp.zeros((_CH, d), jnp.float32)
    return k(zeros, src_i, dst_i, table)


def _sc_count(dst_a, dst_b, ar):
    ea = dst_a.shape[0] - dst_a.shape[0] % _CH
    eb = dst_b.shape[0]
    et = ea + eb
    d = 128
    epw = et // _NW
    steps = epw // _CH
    ngroups = steps // _NBUF
    zr = ar // _NS
    zfull, ztail = zr // _CH, zr % _CH
    mesh = plsc.VectorSubcoreMesh(core_axis_name="c", subcore_axis_name="s")

    @functools.partial(
        pl.kernel,
        out_type=jax.ShapeDtypeStruct((_NC, ar, d), jnp.float32),
        mesh=mesh,
        scratch_types=[
            [pltpu.VMEM((_CH,), jnp.int32)] * _NBUF,
            pltpu.VMEM((_CH, d), jnp.float32),
            pltpu.VMEM_SHARED((ar, d), jnp.float32),
            [pltpu.SemaphoreType.DMA] * _NBUF,
        ],
    )
    def k(zeros_hbm, ones_hbm, dsta_hbm, dstb_hbm, out_hbm,
          idx_d, rows, acc, sem_i):
        c = lax.axis_index("c")
        s = lax.axis_index("s")
        wid = s * _NC + c
        base = pl.multiple_of(wid * epw, _CH)
        pltpu.sync_copy(zeros_hbm, rows)
        zb = s * zr
        for j in range(zfull):
            pltpu.sync_copy(rows, acc.at[pl.ds(zb + j * _CH, _CH)])
        if ztail:
            pltpu.sync_copy(rows.at[pl.ds(0, ztail)],
                            acc.at[pl.ds(zb + zfull * _CH, ztail)])
        plsc.subcore_barrier()

        pltpu.sync_copy(ones_hbm, rows)

        def start_idx(bb, off):
            @pl.when(off < ea)
            def _():
                pltpu.async_copy(dsta_hbm.at[pl.ds(pl.multiple_of(off, _CH),
                                                   _CH)],
                                 idx_d[bb], sem_i[bb])

            @pl.when(off >= ea)
            def _():
                off2 = pl.multiple_of(off - ea, _CH)
                pltpu.async_copy(dstb_hbm.at[pl.ds(off2, _CH)],
                                 idx_d[bb], sem_i[bb])

        for bb in range(_NBUF):
            start_idx(bb, base + bb * _CH)

        def group(g, carry):
            ch0 = g * _NBUF
            for bb in range(_NBUF):
                ch = ch0 + bb
                pltpu.make_async_copy(dstb_hbm.at[pl.ds(0, _CH)],
                                      idx_d[bb], sem_i[bb]).wait()
                pltpu.sync_copy(rows, acc.at[idx_d[bb]], add=True)

                @pl.when(g < ngroups - 1)
                def _():
                    start_idx(bb, base + (ch + _NBUF) * _CH)
            return carry

        lax.fori_loop(0, ngroups, group, 0)
        plsc.subcore_barrier()

        for cc in range(_NC):
            @pl.when(c == cc)
            def _():
                for j in range(zfull):
                    pltpu.sync_copy(acc.at[pl.ds(zb + j * _CH, _CH)],
                                    out_hbm.at[cc, pl.ds(zb + j * _CH, _CH)])
                if ztail:
                    pltpu.sync_copy(acc.at[pl.ds(zb + zfull * _CH, ztail)],
                                    out_hbm.at[cc, pl.ds(zb + zfull * _CH, ztail)])

    zeros = jnp.zeros((_CH, d), jnp.float32)
    ones = jnp.ones((_CH, d), jnp.float32)
    return k(zeros, ones, dst_a, dst_b)


def _tc_mm0(xp, w0):
    np_, in_ch = xp.shape
    hid = w0.shape[1]

    def body(x_ref, w0_ref, y_ref):
        y_ref[...] = jnp.dot(x_ref[...], w0_ref[...],
                             preferred_element_type=jnp.float32)

    return pl.pallas_call(
        body,
        grid=(np_ // _ROWB,),
        in_specs=[
            pl.BlockSpec((_ROWB, in_ch), lambda i: (i, 0)),
            pl.BlockSpec((in_ch, hid), lambda i: (0, 0)),
        ],
        out_specs=pl.BlockSpec((_ROWB, hid), lambda i: (i, 0)),
        out_shape=jax.ShapeDtypeStruct((np_, hid), jnp.float32),
    )(xp, w0)


def _tc_prep(dp, y0, n):
    np_, hid = y0.shape

    def body(dp_ref, y_ref, hh_ref, dinv_ref):
        i = pl.program_id(0)
        degc = (dp_ref[0] + dp_ref[1])[:, 0:1] + 1.0
        rows = i * _ROWB + lax.broadcasted_iota(jnp.int32, (_ROWB, 1), 0)
        dinv = jnp.where(rows < n, lax.rsqrt(degc), 0.0)
        hh_ref[...] = y_ref[...] * dinv
        dinv_ref[...] = dinv

    return pl.pallas_call(
        body,
        grid=(np_ // _ROWB,),
        in_specs=[
            pl.BlockSpec((2, _ROWB, 128), lambda i: (0, i, 0)),
            pl.BlockSpec((_ROWB, hid), lambda i: (i, 0)),
        ],
        out_specs=[
            pl.BlockSpec((_ROWB, hid), lambda i: (i, 0)),
            pl.BlockSpec((_ROWB, 1), lambda i: (i, 0)),
        ],
        out_shape=[
            jax.ShapeDtypeStruct((np_, hid), jnp.float32),
            jax.ShapeDtypeStruct((np_, 1), jnp.float32),
        ],
    )(dp, y0)


def _tc_layer(p, hh, dinv, bias, w):
    np_, hid = hh.shape

    def body(p_ref, hh_ref, dinv_ref, b_ref, w_ref, o_ref):
        dv = dinv_ref[...]
        h = dv * (p_ref[0] + p_ref[1] + hh_ref[...]) + b_ref[...]
        h = jnp.maximum(h, 0.0)
        o_ref[...] = jnp.dot(h, w_ref[...],
                             preferred_element_type=jnp.float32) * dv

    return pl.pallas_call(
        body,
        grid=(np_ // _ROWB,),
        in_specs=[
            pl.BlockSpec((2, _ROWB, hid), lambda i: (0, i, 0)),
            pl.BlockSpec((_ROWB, hid), lambda i: (i, 0)),
            pl.BlockSpec((_ROWB, 1), lambda i: (i, 0)),
            pl.BlockSpec((1, hid), lambda i: (0, 0)),
            pl.BlockSpec((hid, hid), lambda i: (0, 0)),
        ],
        out_specs=pl.BlockSpec((_ROWB, hid), lambda i: (i, 0)),
        out_shape=jax.ShapeDtypeStruct((np_, hid), jnp.float32),
    )(p, hh, dinv, bias, w)


def _tc_combine(p, hh, dinv, bias):
    np_, hid = hh.shape

    def body(p_ref, hh_ref, dinv_ref, b_ref, o_ref):
        o_ref[...] = (dinv_ref[...] * (p_ref[0] + p_ref[1] + hh_ref[...])
                      + b_ref[...])

    return pl.pallas_call(
        body,
        grid=(np_ // _ROWB,),
        in_specs=[
            pl.BlockSpec((2, _ROWB, hid), lambda i: (0, i, 0)),
            pl.BlockSpec((_ROWB, hid), lambda i: (i, 0)),
            pl.BlockSpec((_ROWB, 1), lambda i: (i, 0)),
            pl.BlockSpec((1, hid), lambda i: (0, 0)),
        ],
        out_specs=pl.BlockSpec((_ROWB, hid), lambda i: (i, 0)),
        out_shape=jax.ShapeDtypeStruct((np_, hid), jnp.float32),
    )(p, hh, dinv, bias)


def _tc_head(pp, cp, w1, b1, w2, b2, b, np_, hid):
    pl_ar = pp.shape[1]

    def body(pp_ref, cp_ref, w1_ref, b1_ref, w2_ref, b2_ref, o_ref):
        ps = pp_ref[0] + pp_ref[1]
        cnt = (cp_ref[0] + cp_ref[1])[:, 0:1]
        g = ps[0:b] / jnp.maximum(cnt, 1.0)
        hu = ps[b:2 * b]
        hv = ps[2 * b:3 * b]
        hid_a = (jnp.dot(g, w1_ref[0:hid],
                         preferred_element_type=jnp.float32)
                 + jnp.dot(hu, w1_ref[hid:2 * hid],
                           preferred_element_type=jnp.float32)
                 + jnp.dot(hv, w1_ref[2 * hid:3 * hid],
                           preferred_element_type=jnp.float32)
                 + b1_ref[...])
        hid_a = jnp.maximum(hid_a, 0.0)
        o_ref[...] = jnp.dot(hid_a, w2_ref[...],
                             preferred_element_type=jnp.float32) + b2_ref[...]

    return pl.pallas_call(
        body,
        grid=(1,),
        in_specs=[
            pl.BlockSpec((2, 3 * b, hid), lambda i: (0, 0, 0)),
            pl.BlockSpec((2, b, 128), lambda i: (0, np_ // b, 0)),
            pl.BlockSpec((3 * hid, hid), lambda i: (0, 0)),
            pl.BlockSpec((1, hid), lambda i: (0, 0)),
            pl.BlockSpec((hid, 1), lambda i: (0, 0)),
            pl.BlockSpec((1, 1), lambda i: (0, 0)),
        ],
        out_specs=pl.BlockSpec((b, 1), lambda i: (0, 0)),
        out_shape=jax.ShapeDtypeStruct((b, 1), jnp.float32),
    )(pp, cp, w1, b1, w2, b2)


def kernel(x, edge_index, batch_vec, u_idx, v_idx,
           conv_w0, conv_b0, conv_w1, conv_b1, conv_w2, conv_b2,
           mlp_w1, mlp_b1, mlp_w2, mlp_b2):
    n, in_ch = x.shape
    e = edge_index.shape[1]
    b = u_idx.shape[0]
    hid = conv_w0.shape[1]

    grp = _NW * _CH * _NBUF
    np_ = _rup(n + 1, 1024)
    ep = _rup(e, grp)
    ct_ar = _rup(np_ + b + 1, _CH)
    pe = _rup(n + 2 * b, grp)
    pl_ar = _rup(3 * b + 1, _CH)

    i32 = jnp.int32

    def _pad_rows(npad, lo, hi):
        return lo + jnp.arange(npad, dtype=i32) % (hi - lo)

    ei_p = jnp.concatenate(
        [edge_index, jnp.broadcast_to(_pad_rows(ep - e, n, np_), (2, ep - e))],
        axis=1)
    src_p = ei_p[0]
    dst_p = ei_p[1]
    rem = e % _CH
    ct_e = _rup(e + n, grp)
    cnt_b = jnp.concatenate([
        edge_index[1, e - rem:], np_ + batch_vec,
        _pad_rows(ct_e - e - n, np_ + b, ct_ar)])
    pool_src = jnp.concatenate([
        jnp.arange(n, dtype=i32), u_idx, v_idx,
        _pad_rows(pe - n - 2 * b, n, np_)])
    pool_dst = jnp.concatenate([
        batch_vec, b + jnp.arange(b, dtype=i32), 2 * b + jnp.arange(b, dtype=i32),
        _pad_rows(pe - n - 2 * b, 3 * b, pl_ar)])
    xp = jnp.pad(x, ((0, np_ - n), (0, 0)))

    cp = _sc_count(edge_index[1], cnt_b, ct_ar)
    y0 = _tc_mm0(xp, conv_w0)

    hh0, dinv = _tc_prep(cp, y0, n)

    p0 = _sc_scatter(src_p, dst_p, hh0, np_)
    hh1 = _tc_layer(p0, hh0, dinv, conv_b0.reshape(1, hid), conv_w1)
    p1 = _sc_scatter(src_p, dst_p, hh1, np_)
    hh2 = _tc_layer(p1, hh1, dinv, conv_b1.reshape(1, hid), conv_w2)
    p2 = _sc_scatter(src_p, dst_p, hh2, np_)
    h3 = _tc_combine(p2, hh2, dinv, conv_b2.reshape(1, hid))

    pp = _sc_scatter(pool_src, pool_dst, h3, pl_ar)

    out = _tc_head(pp, cp, mlp_w1, mlp_b1.reshape(1, hid),
                   mlp_w2, mlp_b2.reshape(1, 1), b, np_, hid)
    return out[:, 0]


# speedup vs baseline: 1.0137x; 1.0137x over previous
import functools

import jax
import jax.numpy as jnp
from jax import lax
from jax.experimental import pallas as pl
from jax.experimental.pallas import tpu as pltpu
from jax.experimental.pallas import tpu_sc as plsc

_NC = 2
_NS = 16
_NW = _NC * _NS
_CH = 128
_ROWB = 2048


def _rup(x, m):
    return (x + m - 1) // m * m


_NBUF = 4


def _sc_scatter(src_i, dst_i, table, ar):
    et = src_i.shape[0]
    d = table.shape[1]
    epw = et // _NW
    steps = epw // _CH
    ngroups = steps // _NBUF
    zr = ar // _NS
    zfull, ztail = zr // _CH, zr % _CH
    mesh = plsc.VectorSubcoreMesh(core_axis_name="c", subcore_axis_name="s")

    @functools.partial(
        pl.kernel,
        out_type=jax.ShapeDtypeStruct((_NC, ar, d), jnp.float32),
        mesh=mesh,
        scratch_types=[
            [pltpu.VMEM((_CH,), jnp.int32)] * _NBUF,
            [pltpu.VMEM((_CH,), jnp.int32)] * _NBUF,
            [pltpu.VMEM((_CH, d), jnp.float32)] * 2,
            pltpu.VMEM_SHARED((ar, d), jnp.float32),
            [pltpu.SemaphoreType.DMA] * _NBUF,
            [pltpu.SemaphoreType.DMA] * _NBUF,
            [pltpu.SemaphoreType.DMA] * 2,
        ],
    )
    def k(zeros_hbm, src_hbm, dst_hbm, table_hbm, out_hbm,
          idx_s, idx_d, rows, acc, sem_is, sem_id, sem_g):
        c = lax.axis_index("c")
        s = lax.axis_index("s")
        wid = s * _NC + c
        base = pl.multiple_of(wid * epw, _CH)
        pltpu.sync_copy(zeros_hbm, rows[0])
        zb = s * zr
        for j in range(zfull):
            pltpu.sync_copy(rows[0], acc.at[pl.ds(zb + j * _CH, _CH)])
        if ztail:
            pltpu.sync_copy(rows[0].at[pl.ds(0, ztail)],
                            acc.at[pl.ds(zb + zfull * _CH, ztail)])
        plsc.subcore_barrier()

        def start_idx(dstbuf, hbm, ch, sem):
            off = pl.multiple_of(base + ch * _CH, _CH)
            pltpu.async_copy(hbm.at[pl.ds(off, _CH)], dstbuf, sem)

        def wait_idx(dstbuf, hbm, sem):
            pltpu.make_async_copy(hbm.at[pl.ds(base, _CH)], dstbuf, sem).wait()

        for q in range(_NBUF):
            start_idx(idx_s[q], src_hbm, q, sem_is[q])
            start_idx(idx_d[q], dst_hbm, q, sem_id[q])
        for r in range(2):
            wait_idx(idx_s[r], src_hbm, sem_is[r])
            pltpu.async_copy(table_hbm.at[idx_s[r]], rows[r], sem_g[r])

        def group(g, carry):
            ch0 = g * _NBUF
            for bb in range(_NBUF):
                ch = ch0 + bb
                r = bb % 2
                q = bb
                pltpu.make_async_copy(table_hbm.at[idx_s[q]],
                                      rows[r], sem_g[r]).wait()

                @pl.when(ch + _NBUF < steps)
                def _():
                    start_idx(idx_s[q], src_hbm, ch + _NBUF, sem_is[q])

                wait_idx(idx_d[q], dst_hbm, sem_id[q])
                pltpu.sync_copy(rows[r], acc.at[idx_d[q]], add=True)

                @pl.when(ch + _NBUF < steps)
                def _():
                    start_idx(idx_d[q], dst_hbm, ch + _NBUF, sem_id[q])

                @pl.when(ch + 2 < steps)
                def _():
                    q2 = (bb + 2) % _NBUF
                    wait_idx(idx_s[q2], src_hbm, sem_is[q2])
                    pltpu.async_copy(table_hbm.at[idx_s[q2]], rows[r],
                                     sem_g[r])
            return carry

        lax.fori_loop(0, ngroups, group, 0)
        plsc.subcore_barrier()

        for cc in range(_NC):
            @pl.when(c == cc)
            def _():
                for j in range(zfull):
                    pltpu.sync_copy(acc.at[pl.ds(zb + j * _CH, _CH)],
                                    out_hbm.at[cc, pl.ds(zb + j * _CH, _CH)])
                if ztail:
                    pltpu.sync_copy(acc.at[pl.ds(zb + zfull * _CH, ztail)],
                                    out_hbm.at[cc, pl.ds(zb + zfull * _CH, ztail)])

    zeros = jnp.zeros((_CH, d), jnp.float32)
    return k(zeros, src_i, dst_i, table)


def _sc_count(dst_a, dst_b, ar):
    ea = dst_a.shape[0] - dst_a.shape[0] % _CH
    eb = dst_b.shape[0]
    et = ea + eb
    d = 128
    epw = et // _NW
    steps = epw // _CH
    ngroups = steps // _NBUF
    zr = ar // _NS
    zfull, ztail = zr // _CH, zr % _CH
    mesh = plsc.VectorSubcoreMesh(core_axis_name="c", subcore_axis_name="s")

    @functools.partial(
        pl.kernel,
        out_type=jax.ShapeDtypeStruct((_NC, ar, d), jnp.float32),
        mesh=mesh,
        scratch_types=[
            [pltpu.VMEM((_CH,), jnp.int32)] * _NBUF,
            pltpu.VMEM((_CH, d), jnp.float32),
            pltpu.VMEM_SHARED((ar, d), jnp.float32),
            [pltpu.SemaphoreType.DMA] * _NBUF,
        ],
    )
    def k(zeros_hbm, ones_hbm, dsta_hbm, dstb_hbm, out_hbm,
          idx_d, rows, acc, sem_i):
        c = lax.axis_index("c")
        s = lax.axis_index("s")
        wid = s * _NC + c
        base = pl.multiple_of(wid * epw, _CH)
        pltpu.sync_copy(zeros_hbm, rows)
        zb = s * zr
        for j in range(zfull):
            pltpu.sync_copy(rows, acc.at[pl.ds(zb + j * _CH, _CH)])
        if ztail:
            pltpu.sync_copy(rows.at[pl.ds(0, ztail)],
                            acc.at[pl.ds(zb + zfull * _CH, ztail)])
        plsc.subcore_barrier()

        pltpu.sync_copy(ones_hbm, rows)

        def start_idx(bb, off):
            @pl.when(off < ea)
            def _():
                pltpu.async_copy(dsta_hbm.at[pl.ds(pl.multiple_of(off, _CH),
                                                   _CH)],
                                 idx_d[bb], sem_i[bb])

            @pl.when(off >= ea)
            def _():
                off2 = pl.multiple_of(off - ea, _CH)
                pltpu.async_copy(dstb_hbm.at[pl.ds(off2, _CH)],
                                 idx_d[bb], sem_i[bb])

        for bb in range(_NBUF):
            start_idx(bb, base + bb * _CH)

        def group(g, carry):
            ch0 = g * _NBUF
            for bb in range(_NBUF):
                ch = ch0 + bb
                pltpu.make_async_copy(dstb_hbm.at[pl.ds(0, _CH)],
                                      idx_d[bb], sem_i[bb]).wait()
                pltpu.sync_copy(rows, acc.at[idx_d[bb]], add=True)

                @pl.when(g < ngroups - 1)
                def _():
                    start_idx(bb, base + (ch + _NBUF) * _CH)
            return carry

        lax.fori_loop(0, ngroups, group, 0)
        plsc.subcore_barrier()

        for cc in range(_NC):
            @pl.when(c == cc)
            def _():
                for j in range(zfull):
                    pltpu.sync_copy(acc.at[pl.ds(zb + j * _CH, _CH)],
                                    out_hbm.at[cc, pl.ds(zb + j * _CH, _CH)])
                if ztail:
                    pltpu.sync_copy(acc.at[pl.ds(zb + zfull * _CH, ztail)],
                                    out_hbm.at[cc, pl.ds(zb + zfull * _CH, ztail)])

    zeros = jnp.zeros((_CH, d), jnp.float32)
    ones = jnp.ones((_CH, d), jnp.float32)
    return k(zeros, ones, dst_a, dst_b)


def _tc_mm0(xp, w0):
    np_, in_ch = xp.shape
    hid = w0.shape[1]

    def body(x_ref, w0_ref, y_ref):
        y_ref[...] = jnp.dot(x_ref[...], w0_ref[...],
                             preferred_element_type=jnp.float32)

    return pl.pallas_call(
        body,
        grid=(np_ // _ROWB,),
        in_specs=[
            pl.BlockSpec((_ROWB, in_ch), lambda i: (i, 0)),
            pl.BlockSpec((in_ch, hid), lambda i: (0, 0)),
        ],
        out_specs=pl.BlockSpec((_ROWB, hid), lambda i: (i, 0)),
        out_shape=jax.ShapeDtypeStruct((np_, hid), jnp.float32),
    )(xp, w0)


def _tc_prep(dp, y0, n):
    np_, hid = y0.shape

    def body(dp_ref, y_ref, hh_ref, dinv_ref):
        i = pl.program_id(0)
        degc = (dp_ref[0] + dp_ref[1])[:, 0:1] + 1.0
        rows = i * _ROWB + lax.broadcasted_iota(jnp.int32, (_ROWB, 1), 0)
        dinv = jnp.where(rows < n, lax.rsqrt(degc), 0.0)
        hh_ref[...] = y_ref[...] * dinv
        dinv_ref[...] = dinv

    return pl.pallas_call(
        body,
        grid=(np_ // _ROWB,),
        in_specs=[
            pl.BlockSpec((2, _ROWB, 128), lambda i: (0, i, 0)),
            pl.BlockSpec((_ROWB, hid), lambda i: (i, 0)),
        ],
        out_specs=[
            pl.BlockSpec((_ROWB, hid), lambda i: (i, 0)),
            pl.BlockSpec((_ROWB, 1), lambda i: (i, 0)),
        ],
        out_shape=[
            jax.ShapeDtypeStruct((np_, hid), jnp.float32),
            jax.ShapeDtypeStruct((np_, 1), jnp.float32),
        ],
    )(dp, y0)


def _tc_layer(p, hh, dinv, bias, w):
    np_, hid = hh.shape

    def body(p_ref, hh_ref, dinv_ref, b_ref, w_ref, o_ref):
        dv = dinv_ref[...]
        h = dv * (p_ref[0] + p_ref[1] + hh_ref[...]) + b_ref[...]
        h = jnp.maximum(h, 0.0)
        o_ref[...] = jnp.dot(h, w_ref[...],
                             preferred_element_type=jnp.float32) * dv

    return pl.pallas_call(
        body,
        grid=(np_ // _ROWB,),
        in_specs=[
            pl.BlockSpec((2, _ROWB, hid), lambda i: (0, i, 0)),
            pl.BlockSpec((_ROWB, hid), lambda i: (i, 0)),
            pl.BlockSpec((_ROWB, 1), lambda i: (i, 0)),
            pl.BlockSpec((1, hid), lambda i: (0, 0)),
            pl.BlockSpec((hid, hid), lambda i: (0, 0)),
        ],
        out_specs=pl.BlockSpec((_ROWB, hid), lambda i: (i, 0)),
        out_shape=jax.ShapeDtypeStruct((np_, hid), jnp.float32),
    )(p, hh, dinv, bias, w)


def _tc_combine(p, hh, dinv, bias):
    np_, hid = hh.shape

    def body(p_ref, hh_ref, dinv_ref, b_ref, o_ref):
        o_ref[...] = (dinv_ref[...] * (p_ref[0] + p_ref[1] + hh_ref[...])
                      + b_ref[...])

    return pl.pallas_call(
        body,
        grid=(np_ // _ROWB,),
        in_specs=[
            pl.BlockSpec((2, _ROWB, hid), lambda i: (0, i, 0)),
            pl.BlockSpec((_ROWB, hid), lambda i: (i, 0)),
            pl.BlockSpec((_ROWB, 1), lambda i: (i, 0)),
            pl.BlockSpec((1, hid), lambda i: (0, 0)),
        ],
        out_specs=pl.BlockSpec((_ROWB, hid), lambda i: (i, 0)),
        out_shape=jax.ShapeDtypeStruct((np_, hid), jnp.float32),
    )(p, hh, dinv, bias)


def _tc_head(pp, cp, w1, b1, w2, b2, b, np_, hid):
    pl_ar = pp.shape[1]

    def body(pp_ref, cp_ref, w1_ref, b1_ref, w2_ref, b2_ref, o_ref):
        ps = pp_ref[0] + pp_ref[1]
        cnt = (cp_ref[0] + cp_ref[1])[:, 0:1]
        g = ps[0:b] / jnp.maximum(cnt, 1.0)
        hu = ps[b:2 * b]
        hv = ps[2 * b:3 * b]
        hid_a = (jnp.dot(g, w1_ref[0:hid],
                         preferred_element_type=jnp.float32)
                 + jnp.dot(hu, w1_ref[hid:2 * hid],
                           preferred_element_type=jnp.float32)
                 + jnp.dot(hv, w1_ref[2 * hid:3 * hid],
                           preferred_element_type=jnp.float32)
                 + b1_ref[...])
        hid_a = jnp.maximum(hid_a, 0.0)
        o_ref[...] = jnp.dot(hid_a, w2_ref[...],
                             preferred_element_type=jnp.float32) + b2_ref[...]

    return pl.pallas_call(
        body,
        grid=(1,),
        in_specs=[
            pl.BlockSpec((2, 3 * b, hid), lambda i: (0, 0, 0)),
            pl.BlockSpec((2, b, 128), lambda i: (0, np_ // b, 0)),
            pl.BlockSpec((3 * hid, hid), lambda i: (0, 0)),
            pl.BlockSpec((1, hid), lambda i: (0, 0)),
            pl.BlockSpec((hid, 1), lambda i: (0, 0)),
            pl.BlockSpec((1, 1), lambda i: (0, 0)),
        ],
        out_specs=pl.BlockSpec((b, 1), lambda i: (0, 0)),
        out_shape=jax.ShapeDtypeStruct((b, 1), jnp.float32),
    )(pp, cp, w1, b1, w2, b2)


def kernel(x, edge_index, batch_vec, u_idx, v_idx,
           conv_w0, conv_b0, conv_w1, conv_b1, conv_w2, conv_b2,
           mlp_w1, mlp_b1, mlp_w2, mlp_b2):
    n, in_ch = x.shape
    e = edge_index.shape[1]
    b = u_idx.shape[0]
    hid = conv_w0.shape[1]

    grp = _NW * _CH * _NBUF
    np_ = _rup(n + 1, 1024)
    ep = _rup(e, grp)
    ct_ar = _rup(np_ + b + 1, _CH)
    pe = _rup(n + 2 * b, grp)
    pl_ar = _rup(3 * b + 1, _CH)

    i32 = jnp.int32

    def _pad_rows(npad, lo, hi):
        return lo + jnp.arange(npad, dtype=i32) % (hi - lo)

    ei_p = jnp.concatenate(
        [edge_index, jnp.broadcast_to(_pad_rows(ep - e, n, np_), (2, ep - e))],
        axis=1)
    src_p = ei_p[0]
    dst_p = ei_p[1]
    rem = e % _CH
    ct_e = _rup(e + n, grp)
    cnt_b = jnp.concatenate([
        edge_index[1, e - rem:], np_ + batch_vec,
        _pad_rows(ct_e - e - n, np_ + b, ct_ar)])
    pool_src = jnp.concatenate([
        jnp.arange(n, dtype=i32), u_idx, v_idx,
        _pad_rows(pe - n - 2 * b, n, np_)])
    pool_dst = jnp.concatenate([
        batch_vec, b + jnp.arange(b, dtype=i32), 2 * b + jnp.arange(b, dtype=i32),
        _pad_rows(pe - n - 2 * b, 3 * b, pl_ar)])
    xp = jnp.pad(x, ((0, np_ - n), (0, 0)))

    cp = _sc_count(edge_index[1], cnt_b, ct_ar)
    y0 = _tc_mm0(xp, conv_w0)

    hh0, dinv = _tc_prep(cp, y0, n)

    p0 = _sc_scatter(src_p, dst_p, hh0, np_)
    hh1 = _tc_layer(p0, hh0, dinv, conv_b0.reshape(1, hid), conv_w1)
    p1 = _sc_scatter(src_p, dst_p, hh1, np_)
    hh2 = _tc_layer(p1, hh1, dinv, conv_b1.reshape(1, hid), conv_w2)
    p2 = _sc_scatter(src_p, dst_p, hh2, np_)
    h3 = _tc_combine(p2, hh2, dinv, conv_b2.reshape(1, hid))

    pp = _sc_scatter(pool_src, pool_dst, h3, pl_ar)

    out = _tc_head(pp, cp, mlp_w1, mlp_b1.reshape(1, hid),
                   mlp_w2, mlp_b2.reshape(1, 1), b, np_, hid)
    return out[:, 0]
